# baseline (speedup 1.0000x reference)
import functools

import jax
import jax.numpy as jnp
from jax import lax
from jax.experimental import pallas as pl
from jax.experimental.pallas import tpu as pltpu
from jax.experimental.pallas import tpu_sc as plsc

B = 2048
D = 1024
H = 2048
E = 8
K = 2

T = 256
MAX_TILES = (K * B) // T + E
NPAD = MAX_TILES * T

NW = 32
NPW = NPAD // NW
GCH = 64
TPW = B // NW
GCH2 = 32

BT = 256


def _router_body(t_ref, x_ref, w1_ref, b1_ref, w2_ref, b2_ref,
                 eidx_ref, alph_ref):
    xb = x_ref[...].astype(jnp.bfloat16)
    w1b = w1_ref[:D, :].astype(jnp.bfloat16)
    h = jnp.dot(xb, w1b, preferred_element_type=jnp.float32)
    tb = t_ref[...].astype(jnp.bfloat16).astype(jnp.float32)
    wtb = w1_ref[D, :].astype(jnp.bfloat16).astype(jnp.float32)
    h += tb * wtb[None, :]
    h += b1_ref[...]
    h = h * jax.nn.sigmoid(h)
    lg = jnp.dot(h.astype(jnp.bfloat16), w2_ref[...].astype(jnp.bfloat16),
                 preferred_element_type=jnp.float32)
    lg += b2_ref[...]
    col = lax.broadcasted_iota(jnp.int32, lg.shape, 1)
    m1 = jnp.max(lg, axis=1, keepdims=True)
    i1 = jnp.min(jnp.where(lg == m1, col, E), axis=1, keepdims=True)
    lg2 = jnp.where(col == i1, -jnp.inf, lg)
    m2 = jnp.max(lg2, axis=1, keepdims=True)
    i2 = jnp.min(jnp.where(lg2 == m2, col, E), axis=1, keepdims=True)
    s = jnp.exp(m2 - m1)
    a1 = 1.0 / (1.0 + s)
    a2 = s / (1.0 + s)
    eidx_ref[...] = jnp.where(col == 0, i1, jnp.where(col == 1, i2, 0))
    alph_ref[...] = jnp.where(col == 0, a1, jnp.where(col == 1, a2, 0.0))


def _router(t_col, x, rW1, rb1, rW2, rb2):
    return pl.pallas_call(
        _router_body,
        grid=(B // BT,),
        in_specs=[
            pl.BlockSpec((BT, 1), lambda i: (i, 0)),
            pl.BlockSpec((BT, D), lambda i: (i, 0)),
            pl.BlockSpec((D + 1, H), lambda i: (0, 0)),
            pl.BlockSpec((1, H), lambda i: (0, 0)),
            pl.BlockSpec((H, E), lambda i: (0, 0)),
            pl.BlockSpec((1, E), lambda i: (0, 0)),
        ],
        out_specs=[
            pl.BlockSpec((BT, E), lambda i: (i, 0)),
            pl.BlockSpec((BT, E), lambda i: (i, 0)),
        ],
        out_shape=[
            jax.ShapeDtypeStruct((B, E), jnp.int32),
            jax.ShapeDtypeStruct((B, E), jnp.float32),
        ],
        compiler_params=pltpu.CompilerParams(
            dimension_semantics=("arbitrary",)),
    )(t_col, x, rW1, rb1.reshape(1, H), rW2, rb2.reshape(1, E))


def _g1_body(x_hbm, t128_hbm, pos1_hbm, pos2_hbm, xs_hbm, ts_hbm,
             ip1_v, ip2_v, rows_v, trows_v, sem1, sem2, sem3, sem4):
    wid = lax.axis_index("s") * 2 + lax.axis_index("c")
    base = wid * TPW
    pltpu.sync_copy(pos1_hbm.at[pl.ds(base, TPW)], ip1_v)
    pltpu.sync_copy(pos2_hbm.at[pl.ds(base, TPW)], ip2_v)
    pltpu.sync_copy(x_hbm.at[pl.ds(base, TPW)], rows_v)
    pltpu.sync_copy(t128_hbm.at[pl.ds(base, TPW)], trows_v)
    cp1 = pltpu.async_copy(rows_v, xs_hbm.at[ip1_v], sem1)
    cp2 = pltpu.async_copy(rows_v, xs_hbm.at[ip2_v], sem2)
    cp3 = pltpu.async_copy(trows_v, ts_hbm.at[ip1_v], sem3)
    cp4 = pltpu.async_copy(trows_v, ts_hbm.at[ip2_v], sem4)
    cp1.wait()
    cp2.wait()
    cp3.wait()
    cp4.wait()


def _g1(x, t128, pos1, pos2):
    mesh = plsc.VectorSubcoreMesh(core_axis_name="c", subcore_axis_name="s")
    fn = pl.kernel(
        _g1_body,
        out_type=[
            jax.ShapeDtypeStruct((NPAD, D), jnp.float32),
            jax.ShapeDtypeStruct((NPAD, 128), jnp.float32),
        ],
        mesh=mesh,
        scratch_types=[
            pltpu.VMEM((TPW,), jnp.int32),
            pltpu.VMEM((TPW,), jnp.int32),
            pltpu.VMEM((TPW, D), jnp.float32),
            pltpu.VMEM((TPW, 128), jnp.float32),
            pltpu.SemaphoreType.DMA,
            pltpu.SemaphoreType.DMA,
            pltpu.SemaphoreType.DMA,
            pltpu.SemaphoreType.DMA,
        ],
    )
    return fn(x, t128, pos1, pos2)


def _l1_body(te_ref, nt_ref, xs_ref, ts_ref, w1_ref, b1_ref, out_ref):
    i = pl.program_id(0)

    @pl.when(i * T < nt_ref[0])
    def _():
        acc = jnp.dot(xs_ref[...].astype(jnp.bfloat16),
                      w1_ref[0, :D, :].astype(jnp.bfloat16),
                      preferred_element_type=jnp.float32)
        tsb = ts_ref[:, 0:1].astype(jnp.bfloat16).astype(jnp.float32)
        wtb = w1_ref[0, D, :].astype(jnp.bfloat16).astype(jnp.float32)
        acc += tsb * wtb[None, :]
        acc += b1_ref[0]
        out_ref[...] = acc * jax.nn.sigmoid(acc)


def _l2_body(te_ref, nt_ref, y1_ref, w2_ref, b2_ref, out_ref):
    i = pl.program_id(0)

    @pl.when(i * T < nt_ref[0])
    def _():
        acc = jnp.dot(y1_ref[...].astype(jnp.bfloat16),
                      w2_ref[0].astype(jnp.bfloat16),
                      preferred_element_type=jnp.float32)
        acc += b2_ref[0]
        out_ref[...] = acc * jax.nn.sigmoid(acc)


def _l3_body(te_ref, nt_ref, y2_ref, w3_ref, b3_ref, out_ref):
    i = pl.program_id(0)

    @pl.when(i * T < nt_ref[0])
    def _():
        acc = jnp.dot(y2_ref[...].astype(jnp.bfloat16),
                      w3_ref[0].astype(jnp.bfloat16),
                      preferred_element_type=jnp.float32)
        acc += b3_ref[0]
        out_ref[...] = acc


def _expert_mlp(te, nt, xs, ts_col, We1, be1, We2, be2, We3, be3):
    params = pltpu.CompilerParams(dimension_semantics=("arbitrary",))
    ys1 = pl.pallas_call(
        _l1_body,
        grid_spec=pltpu.PrefetchScalarGridSpec(
            num_scalar_prefetch=2,
            grid=(MAX_TILES,),
            in_specs=[
                pl.BlockSpec((T, D), lambda i, te, nt: (i, 0)),
                pl.BlockSpec((T, 128), lambda i, te, nt: (i, 0)),
                pl.BlockSpec((1, D + 1, H), lambda i, te, nt: (te[i], 0, 0)),
                pl.BlockSpec((1, 1, H), lambda i, te, nt: (te[i], 0, 0)),
            ],
            out_specs=pl.BlockSpec((T, H), lambda i, te, nt: (i, 0)),
        ),
        out_shape=jax.ShapeDtypeStruct((NPAD, H), jnp.float32),
        compiler_params=params,
    )(te, nt, xs, ts_col, We1, be1.reshape(E, 1, H))
    ys2 = pl.pallas_call(
        _l2_body,
        grid_spec=pltpu.PrefetchScalarGridSpec(
            num_scalar_prefetch=2,
            grid=(MAX_TILES,),
            in_specs=[
                pl.BlockSpec((T, H), lambda i, te, nt: (i, 0)),
                pl.BlockSpec((1, H, H), lambda i, te, nt: (te[i], 0, 0)),
                pl.BlockSpec((1, 1, H), lambda i, te, nt: (te[i], 0, 0)),
            ],
            out_specs=pl.BlockSpec((T, H), lambda i, te, nt: (i, 0)),
        ),
        out_shape=jax.ShapeDtypeStruct((NPAD, H), jnp.float32),
        compiler_params=params,
    )(te, nt, ys1, We2, be2.reshape(E, 1, H))
    ysw = pl.pallas_call(
        _l3_body,
        grid_spec=pltpu.PrefetchScalarGridSpec(
            num_scalar_prefetch=2,
            grid=(MAX_TILES,),
            in_specs=[
                pl.BlockSpec((T, H), lambda i, te, nt: (i, 0)),
                pl.BlockSpec((1, H, D), lambda i, te, nt: (te[i], 0, 0)),
                pl.BlockSpec((1, 1, D), lambda i, te, nt: (te[i], 0, 0)),
            ],
            out_specs=pl.BlockSpec((T, D), lambda i, te, nt: (i, 0)),
        ),
        out_shape=jax.ShapeDtypeStruct((NPAD, D), jnp.float32),
        compiler_params=params,
    )(te, nt, ys2, We3, be3.reshape(E, 1, D))
    return ysw


def _g2_body(ys_hbm, p1_hbm, p2_hbm, y1_hbm, y2_hbm,
             idx1_v, idx2_v, rows1_v, rows2_v, sem1, sem2):
    wid = lax.axis_index("s") * 2 + lax.axis_index("c")
    for c in range(TPW // GCH2):
        b0 = wid * TPW + c * GCH2
        pltpu.sync_copy(p1_hbm.at[pl.ds(b0, GCH2)], idx1_v)
        pltpu.sync_copy(p2_hbm.at[pl.ds(b0, GCH2)], idx2_v)
        cp1 = pltpu.async_copy(ys_hbm.at[idx1_v], rows1_v, sem1)
        cp2 = pltpu.async_copy(ys_hbm.at[idx2_v], rows2_v, sem2)
        cp1.wait()
        cp2.wait()
        pltpu.sync_copy(rows1_v, y1_hbm.at[pl.ds(b0, GCH2)])
        pltpu.sync_copy(rows2_v, y2_hbm.at[pl.ds(b0, GCH2)])


def _g2(ysw, p1, p2):
    mesh = plsc.VectorSubcoreMesh(core_axis_name="c", subcore_axis_name="s")
    fn = pl.kernel(
        _g2_body,
        out_type=[
            jax.ShapeDtypeStruct((B, D), jnp.float32),
            jax.ShapeDtypeStruct((B, D), jnp.float32),
        ],
        mesh=mesh,
        scratch_types=[
            pltpu.VMEM((GCH2,), jnp.int32),
            pltpu.VMEM((GCH2,), jnp.int32),
            pltpu.VMEM((GCH2, D), jnp.float32),
            pltpu.VMEM((GCH2, D), jnp.float32),
            pltpu.SemaphoreType.DMA,
            pltpu.SemaphoreType.DMA,
        ],
    )
    return fn(ysw, p1, p2)


def _add_body(a_ref, b_ref, wa_ref, wb_ref, o_ref):
    o_ref[...] = a_ref[...] * wa_ref[...] + b_ref[...] * wb_ref[...]


def _combine(y1g, y2g, a1c, a2c):
    return pl.pallas_call(
        _add_body,
        grid=(B // 512,),
        in_specs=[
            pl.BlockSpec((512, D), lambda i: (i, 0)),
            pl.BlockSpec((512, D), lambda i: (i, 0)),
            pl.BlockSpec((512, 1), lambda i: (i, 0)),
            pl.BlockSpec((512, 1), lambda i: (i, 0)),
        ],
        out_specs=pl.BlockSpec((512, D), lambda i: (i, 0)),
        out_shape=jax.ShapeDtypeStruct((B, D), jnp.float32),
    )(y1g, y2g, a1c, a2c)


def _route_metadata(e1, e2):
    ei = jnp.concatenate([e1, e2])
    oh = (ei[:, None] == jnp.arange(E)[None, :]).astype(jnp.int32)
    cnt = oh.sum(axis=0)
    rank = jnp.sum((jnp.cumsum(oh, axis=0) - oh) * oh, axis=1)
    pad_cnt = ((cnt + T - 1) // T) * T
    bound = jnp.cumsum(pad_cnt)
    pad_off = bound - pad_cnt
    pos = (jnp.sum(oh * pad_off[None, :], axis=1) + rank).astype(jnp.int32)
    te = jnp.sum((jnp.arange(MAX_TILES, dtype=jnp.int32)[:, None] * T
                  >= bound[None, :]).astype(jnp.int32), axis=1)
    te = jnp.minimum(te, E - 1).astype(jnp.int32)
    nt = bound[-1:].astype(jnp.int32)
    return pos[:B], pos[B:], te, nt


def kernel(t, x, rW1, rb1, rW2, rb2, We1, be1, We2, be2, We3, be3):
    t_col = t.reshape(B, 1)
    eidx, alph = _router(t_col, x, rW1, rb1, rW2, rb2)
    e1, e2 = eidx[:, 0], eidx[:, 1]
    p1, p2, te, nt = _route_metadata(e1, e2)
    p1 = jnp.arange(B, dtype=jnp.int32)
    p2 = jnp.arange(B, dtype=jnp.int32) + B
    te = jnp.arange(MAX_TILES, dtype=jnp.int32) % E
    nt = jnp.full((1,), NPAD, jnp.int32)
    t128 = jnp.broadcast_to(t.reshape(B, 1), (B, 128))
    xs, ts128 = _g1(x, t128, p1, p2)
    ysw = _expert_mlp(te, nt, xs, ts128, We1, be1, We2, be2, We3, be3)
    y1g, y2g = _g2(ysw, p1, p2)
    return _combine(y1g, y2g, alph[:, 0:1], alph[:, 1:2])


# speedup vs baseline: 1.3218x; 1.3218x over previous
import functools

import jax
import jax.numpy as jnp
from jax import lax
from jax.experimental import pallas as pl
from jax.experimental.pallas import tpu as pltpu
from jax.experimental.pallas import tpu_sc as plsc

B = 2048
D = 1024
H = 2048
E = 8
K = 2

T = 256
MAX_TILES = (K * B) // T + E
NPAD = MAX_TILES * T

NW = 32
NPW = NPAD // NW
GCH = 64
TPW = B // NW
GCH2 = 32

BT = 256


def _router_body(t_ref, x_ref, w1_ref, b1_ref, w2_ref, b2_ref,
                 eidx_ref, alph_ref):
    xb = x_ref[...].astype(jnp.bfloat16)
    w1b = w1_ref[:D, :].astype(jnp.bfloat16)
    h = jnp.dot(xb, w1b, preferred_element_type=jnp.float32)
    tb = t_ref[...].astype(jnp.bfloat16).astype(jnp.float32)
    wtb = w1_ref[D, :].astype(jnp.bfloat16).astype(jnp.float32)
    h += tb * wtb[None, :]
    h += b1_ref[...]
    h = h * jax.nn.sigmoid(h)
    lg = jnp.dot(h.astype(jnp.bfloat16), w2_ref[...].astype(jnp.bfloat16),
                 preferred_element_type=jnp.float32)
    lg += b2_ref[...]
    col = lax.broadcasted_iota(jnp.int32, lg.shape, 1)
    m1 = jnp.max(lg, axis=1, keepdims=True)
    i1 = jnp.min(jnp.where(lg == m1, col, E), axis=1, keepdims=True)
    lg2 = jnp.where(col == i1, -jnp.inf, lg)
    m2 = jnp.max(lg2, axis=1, keepdims=True)
    i2 = jnp.min(jnp.where(lg2 == m2, col, E), axis=1, keepdims=True)
    s = jnp.exp(m2 - m1)
    a1 = 1.0 / (1.0 + s)
    a2 = s / (1.0 + s)
    eidx_ref[...] = jnp.where(col == 0, i1, jnp.where(col == 1, i2, 0))
    alph_ref[...] = jnp.where(col == 0, a1, jnp.where(col == 1, a2, 0.0))


def _router(t_col, x, rW1, rb1, rW2, rb2):
    return pl.pallas_call(
        _router_body,
        grid=(B // BT,),
        in_specs=[
            pl.BlockSpec((BT, 1), lambda i: (i, 0)),
            pl.BlockSpec((BT, D), lambda i: (i, 0)),
            pl.BlockSpec((D + 1, H), lambda i: (0, 0)),
            pl.BlockSpec((1, H), lambda i: (0, 0)),
            pl.BlockSpec((H, E), lambda i: (0, 0)),
            pl.BlockSpec((1, E), lambda i: (0, 0)),
        ],
        out_specs=[
            pl.BlockSpec((BT, E), lambda i: (i, 0)),
            pl.BlockSpec((BT, E), lambda i: (i, 0)),
        ],
        out_shape=[
            jax.ShapeDtypeStruct((B, E), jnp.int32),
            jax.ShapeDtypeStruct((B, E), jnp.float32),
        ],
        compiler_params=pltpu.CompilerParams(
            dimension_semantics=("arbitrary",)),
    )(t_col, x, rW1, rb1.reshape(1, H), rW2, rb2.reshape(1, E))


def _g1_body(x_hbm, t128_hbm, pos1_hbm, pos2_hbm, xs_hbm, ts_hbm,
             ip1_v, ip2_v, rows_v, trows_v, sem1, sem2, sem3, sem4):
    wid = lax.axis_index("s") * 2 + lax.axis_index("c")
    base = wid * TPW
    pltpu.sync_copy(pos1_hbm.at[pl.ds(base, TPW)], ip1_v)
    pltpu.sync_copy(pos2_hbm.at[pl.ds(base, TPW)], ip2_v)
    pltpu.sync_copy(x_hbm.at[pl.ds(base, TPW)], rows_v)
    pltpu.sync_copy(t128_hbm.at[pl.ds(base, TPW)], trows_v)
    cp1 = pltpu.async_copy(rows_v, xs_hbm.at[ip1_v], sem1)
    cp2 = pltpu.async_copy(rows_v, xs_hbm.at[ip2_v], sem2)
    cp3 = pltpu.async_copy(trows_v, ts_hbm.at[ip1_v], sem3)
    cp4 = pltpu.async_copy(trows_v, ts_hbm.at[ip2_v], sem4)
    cp1.wait()
    cp2.wait()
    cp3.wait()
    cp4.wait()


def _g1(x, t128, pos1, pos2):
    mesh = plsc.VectorSubcoreMesh(core_axis_name="c", subcore_axis_name="s")
    fn = pl.kernel(
        _g1_body,
        out_type=[
            jax.ShapeDtypeStruct((NPAD, D), jnp.float32),
            jax.ShapeDtypeStruct((NPAD, 128), jnp.float32),
        ],
        mesh=mesh,
        scratch_types=[
            pltpu.VMEM((TPW,), jnp.int32),
            pltpu.VMEM((TPW,), jnp.int32),
            pltpu.VMEM((TPW, D), jnp.float32),
            pltpu.VMEM((TPW, 128), jnp.float32),
            pltpu.SemaphoreType.DMA,
            pltpu.SemaphoreType.DMA,
            pltpu.SemaphoreType.DMA,
            pltpu.SemaphoreType.DMA,
        ],
    )
    return fn(x, t128, pos1, pos2)


def _l1_body(te_ref, nt_ref, xs_ref, ts_ref, w1_ref, b1_ref, out_ref):
    i = pl.program_id(0)

    @pl.when(i * T < nt_ref[0])
    def _():
        acc = jnp.dot(xs_ref[...].astype(jnp.bfloat16),
                      w1_ref[0, :D, :].astype(jnp.bfloat16),
                      preferred_element_type=jnp.float32)
        tsb = ts_ref[:, 0:1].astype(jnp.bfloat16).astype(jnp.float32)
        wtb = w1_ref[0, D, :].astype(jnp.bfloat16).astype(jnp.float32)
        acc += tsb * wtb[None, :]
        acc += b1_ref[0]
        out_ref[...] = acc * jax.nn.sigmoid(acc)


def _l23_body(te_ref, nt_ref, y1_ref, w2_ref, b2_ref, w3_ref, b3_ref,
              out_ref):
    i = pl.program_id(0)

    @pl.when(i * T < nt_ref[0])
    def _():
        acc = jnp.dot(y1_ref[...].astype(jnp.bfloat16),
                      w2_ref[0].astype(jnp.bfloat16),
                      preferred_element_type=jnp.float32)
        acc += b2_ref[0]
        h2 = acc * jax.nn.sigmoid(acc)
        acc3 = jnp.dot(h2.astype(jnp.bfloat16),
                       w3_ref[0].astype(jnp.bfloat16),
                       preferred_element_type=jnp.float32)
        acc3 += b3_ref[0]
        out_ref[...] = acc3


def _expert_mlp(te, nt, xs, ts_col, We1, be1, We2, be2, We3, be3):
    params = pltpu.CompilerParams(dimension_semantics=("arbitrary",))
    ys1 = pl.pallas_call(
        _l1_body,
        grid_spec=pltpu.PrefetchScalarGridSpec(
            num_scalar_prefetch=2,
            grid=(MAX_TILES,),
            in_specs=[
                pl.BlockSpec((T, D), lambda i, te, nt: (i, 0)),
                pl.BlockSpec((T, 128), lambda i, te, nt: (i, 0)),
                pl.BlockSpec((1, D + 1, H), lambda i, te, nt: (te[i], 0, 0)),
                pl.BlockSpec((1, 1, H), lambda i, te, nt: (te[i], 0, 0)),
            ],
            out_specs=pl.BlockSpec((T, H), lambda i, te, nt: (i, 0)),
        ),
        out_shape=jax.ShapeDtypeStruct((NPAD, H), jnp.float32),
        compiler_params=params,
    )(te, nt, xs, ts_col, We1, be1.reshape(E, 1, H))
    ysw = pl.pallas_call(
        _l23_body,
        grid_spec=pltpu.PrefetchScalarGridSpec(
            num_scalar_prefetch=2,
            grid=(MAX_TILES,),
            in_specs=[
                pl.BlockSpec((T, H), lambda i, te, nt: (i, 0)),
                pl.BlockSpec((1, H, H), lambda i, te, nt: (te[i], 0, 0)),
                pl.BlockSpec((1, 1, H), lambda i, te, nt: (te[i], 0, 0)),
                pl.BlockSpec((1, H, D), lambda i, te, nt: (te[i], 0, 0)),
                pl.BlockSpec((1, 1, D), lambda i, te, nt: (te[i], 0, 0)),
            ],
            out_specs=pl.BlockSpec((T, D), lambda i, te, nt: (i, 0)),
        ),
        out_shape=jax.ShapeDtypeStruct((NPAD, D), jnp.float32),
        compiler_params=params,
    )(te, nt, ys1, We2, be2.reshape(E, 1, H), We3, be3.reshape(E, 1, D))
    return ysw


def _g2_body(ys_hbm, p1_hbm, p2_hbm, y1_hbm, y2_hbm,
             idx1_v, idx2_v, rows1_v, rows2_v, sem1, sem2):
    wid = lax.axis_index("s") * 2 + lax.axis_index("c")
    for c in range(TPW // GCH2):
        b0 = wid * TPW + c * GCH2
        pltpu.sync_copy(p1_hbm.at[pl.ds(b0, GCH2)], idx1_v)
        pltpu.sync_copy(p2_hbm.at[pl.ds(b0, GCH2)], idx2_v)
        cp1 = pltpu.async_copy(ys_hbm.at[idx1_v], rows1_v, sem1)
        cp2 = pltpu.async_copy(ys_hbm.at[idx2_v], rows2_v, sem2)
        cp1.wait()
        cp2.wait()
        pltpu.sync_copy(rows1_v, y1_hbm.at[pl.ds(b0, GCH2)])
        pltpu.sync_copy(rows2_v, y2_hbm.at[pl.ds(b0, GCH2)])


def _g2(ysw, p1, p2):
    mesh = plsc.VectorSubcoreMesh(core_axis_name="c", subcore_axis_name="s")
    fn = pl.kernel(
        _g2_body,
        out_type=[
            jax.ShapeDtypeStruct((B, D), jnp.float32),
            jax.ShapeDtypeStruct((B, D), jnp.float32),
        ],
        mesh=mesh,
        scratch_types=[
            pltpu.VMEM((GCH2,), jnp.int32),
            pltpu.VMEM((GCH2,), jnp.int32),
            pltpu.VMEM((GCH2, D), jnp.float32),
            pltpu.VMEM((GCH2, D), jnp.float32),
            pltpu.SemaphoreType.DMA,
            pltpu.SemaphoreType.DMA,
        ],
    )
    return fn(ysw, p1, p2)


def _add_body(a_ref, b_ref, wa_ref, wb_ref, o_ref):
    o_ref[...] = a_ref[...] * wa_ref[...] + b_ref[...] * wb_ref[...]


def _combine(y1g, y2g, a1c, a2c):
    return pl.pallas_call(
        _add_body,
        grid=(B // 512,),
        in_specs=[
            pl.BlockSpec((512, D), lambda i: (i, 0)),
            pl.BlockSpec((512, D), lambda i: (i, 0)),
            pl.BlockSpec((512, 1), lambda i: (i, 0)),
            pl.BlockSpec((512, 1), lambda i: (i, 0)),
        ],
        out_specs=pl.BlockSpec((512, D), lambda i: (i, 0)),
        out_shape=jax.ShapeDtypeStruct((B, D), jnp.float32),
    )(y1g, y2g, a1c, a2c)


def _route_metadata(e1, e2):
    ei = jnp.concatenate([e1, e2])
    oh = (ei[:, None] == jnp.arange(E)[None, :]).astype(jnp.int32)
    cnt = oh.sum(axis=0)
    rank = jnp.sum((jnp.cumsum(oh, axis=0) - oh) * oh, axis=1)
    pad_cnt = ((cnt + T - 1) // T) * T
    bound = jnp.cumsum(pad_cnt)
    pad_off = bound - pad_cnt
    pos = (jnp.sum(oh * pad_off[None, :], axis=1) + rank).astype(jnp.int32)
    te = jnp.sum((jnp.arange(MAX_TILES, dtype=jnp.int32)[:, None] * T
                  >= bound[None, :]).astype(jnp.int32), axis=1)
    te = jnp.minimum(te, E - 1).astype(jnp.int32)
    nt = bound[-1:].astype(jnp.int32)
    return pos[:B], pos[B:], te, nt


def kernel(t, x, rW1, rb1, rW2, rb2, We1, be1, We2, be2, We3, be3):
    t_col = t.reshape(B, 1)
    eidx, alph = _router(t_col, x, rW1, rb1, rW2, rb2)
    e1, e2 = eidx[:, 0], eidx[:, 1]
    p1, p2, te, nt = _route_metadata(e1, e2)
    t128 = jnp.broadcast_to(t.reshape(B, 1), (B, 128))
    xs, ts128 = _g1(x, t128, p1, p2)
    ysw = _expert_mlp(te, nt, xs, ts128, We1, be1, We2, be2, We3, be3)
    y1g, y2g = _g2(ysw, p1, p2)
    return _combine(y1g, y2g, alph[:, 0:1], alph[:, 1:2])
